# baseline (speedup 1.0000x reference)
import functools

import jax
import jax.numpy as jnp
from jax import lax
from jax.experimental import pallas as pl
from jax.experimental.pallas import tpu as pltpu
from jax.experimental.pallas import tpu_sc as plsc

NUM_CLASSES = 1000
FEAT_DIM = 2048
N = 16384
SIGMA = 0.2

L = 16
NUM_TILES = 16
NUM_WORKERS = 32
PASS_COLS = FEAT_DIM // NUM_WORKERS
CGROUPS = PASS_COLS // L
R = 64
RB = R // 8
NCHUNKS = N // R
LGROUPS = R // L
CBLOCKS = FEAT_DIM // 128
BLEND_CHUNK = 200
BLEND_RB = BLEND_CHUNK // 8
BLEND_CHUNKS = NUM_CLASSES // BLEND_CHUNK

_mesh = plsc.VectorSubcoreMesh(core_axis_name="c", subcore_axis_name="s")


@functools.partial(
    pl.kernel,
    out_type=(
        jax.ShapeDtypeStruct((NUM_CLASSES // 8, CBLOCKS, 8, 128), jnp.float32),
        jax.ShapeDtypeStruct((NUM_CLASSES // 8, CBLOCKS, 8, 128), jnp.float32),
    ),
    mesh=_mesh,
    compiler_params=pltpu.CompilerParams(
        use_tc_tiling_on_sc=False, needs_layout_passes=False
    ),
    scratch_types=[
        pltpu.VMEM((NUM_CLASSES, PASS_COLS), jnp.float32),
        pltpu.VMEM((NUM_CLASSES, L), jnp.float32),
        pltpu.VMEM((RB, 8, PASS_COLS), jnp.float32),
        pltpu.VMEM((RB, 8, PASS_COLS), jnp.float32),
        pltpu.VMEM((N,), jnp.int32),
        pltpu.VMEM((BLEND_RB, 8, PASS_COLS), jnp.float32),
        pltpu.SemaphoreType.DMA,
        pltpu.SemaphoreType.DMA,
    ],
)
def _cma_update(
    rgb_hbm, rgb_lab_hbm, vis_mem_hbm, ir_hbm, ir_lab_hbm, ir_mem_hbm,
    vis_out_hbm, ir_out_hbm,
    acc, cnt, feat_a, feat_b, labv, memv, sem_a, sem_b,
):
    c = lax.axis_index("c")
    s = lax.axis_index("s")
    w = c * NUM_TILES + s
    cb = w // 2
    hoff = (w % 2) * PASS_COLS
    iota = lax.iota(jnp.int32, L)
    ones = jnp.ones((L,), jnp.float32)

    def process_chunk(buf, i):
        @plsc.parallel_loop(0, LGROUPS)
        def _(g):
            lvec = labv[pl.ds((i * LGROUPS + g) * L, L)]

            def ld(j):
                lab_bc = jnp.take_along_axis(
                    lvec, jnp.full((L,), j, jnp.int32), axis=0
                )
                b = 2 * g + j // 8
                r = j % 8
                vals = [buf[b, r, pl.ds(cg * L, L)] for cg in range(CGROUPS)]
                return lab_bc, vals

            def st(lab_bc, vals):
                for cg in range(CGROUPS):
                    plsc.addupdate_scatter(
                        acc, [lab_bc, cg * L + iota], vals[cg]
                    )
                plsc.addupdate_scatter(cnt, [lab_bc, iota], ones)

            p0 = ld(0)
            p1 = ld(1)
            for j in range(2, L):
                st(*p0)
                p0 = p1
                p1 = ld(j)
            st(*p0)
            st(*p1)

    for feats_hbm, lab_hbm, mem_hbm, out_hbm in (
        (rgb_hbm, rgb_lab_hbm, vis_mem_hbm, vis_out_hbm),
        (ir_hbm, ir_lab_hbm, ir_mem_hbm, ir_out_hbm),
    ):
        @pl.loop(0, NUM_CLASSES)
        def _(r):
            for cg in range(CGROUPS):
                acc[r, pl.ds(cg * L, L)] = jnp.zeros((L,), jnp.float32)
            cnt[r, :] = jnp.zeros((L,), jnp.float32)

        pltpu.sync_copy(lab_hbm, labv)

        def rows(i):
            return feats_hbm.at[pl.ds(i * RB, RB), cb, :, pl.ds(hoff, PASS_COLS)]

        pltpu.async_copy(rows(0), feat_a, sem_a)

        @pl.loop(0, NCHUNKS // 2)
        def _(i):
            i0 = 2 * i
            i1 = i0 + 1
            pltpu.make_async_copy(rows(i0), feat_a, sem_a).wait()
            pltpu.async_copy(rows(i1), feat_b, sem_b)
            process_chunk(feat_a, i0)
            pltpu.make_async_copy(rows(i1), feat_b, sem_b).wait()

            @pl.when(i1 + 1 < NCHUNKS)
            def _():
                pltpu.async_copy(rows(i1 + 1), feat_a, sem_a)

            process_chunk(feat_b, i1)

        @pl.loop(0, BLEND_CHUNKS)
        def _(ch):
            pltpu.sync_copy(
                mem_hbm.at[
                    pl.ds(ch * BLEND_RB, BLEND_RB), cb, :, pl.ds(hoff, PASS_COLS)
                ],
                memv,
            )

            @pl.loop(0, BLEND_CHUNK)
            def _(jj):
                cls = ch * BLEND_CHUNK + jj
                b = jj // 8
                r = jj % 8
                cvec = cnt[cls, :]
                present = cvec > 0.0
                avec = jnp.where(present, SIGMA / jnp.maximum(cvec, 1.0), 0.0)
                bvec = jnp.where(present, 1.0 - SIGMA, 1.0)
                for cg in range(CGROUPS):
                    sl = pl.ds(cg * L, L)
                    memv[b, r, sl] = bvec * memv[b, r, sl] + avec * acc[cls, sl]

            pltpu.sync_copy(
                memv,
                out_hbm.at[
                    pl.ds(ch * BLEND_RB, BLEND_RB), cb, :, pl.ds(hoff, PASS_COLS)
                ],
            )


def kernel(rgb_feats, ir_feats, rgb_labels, ir_labels, vis_memory, ir_memory):
    def to4(x, nrows):
        return x.reshape(nrows // 8, 8, CBLOCKS, 128).transpose(0, 2, 1, 3)

    def from4(x, nrows):
        return x.transpose(0, 2, 1, 3).reshape(nrows, FEAT_DIM)

    vis_out, ir_out = _cma_update(
        to4(rgb_feats, N),
        rgb_labels.astype(jnp.int32),
        to4(vis_memory, NUM_CLASSES),
        to4(ir_feats, N),
        ir_labels.astype(jnp.int32),
        to4(ir_memory, NUM_CLASSES),
    )
    return (from4(vis_out, NUM_CLASSES), from4(ir_out, NUM_CLASSES))


# speedup vs baseline: 1.3915x; 1.3915x over previous
import functools

import jax
import jax.numpy as jnp
from jax import lax
from jax.experimental import pallas as pl
from jax.experimental.pallas import tpu as pltpu
from jax.experimental.pallas import tpu_sc as plsc

NUM_CLASSES = 1000
FEAT_DIM = 2048
N = 16384
SIGMA = 0.2

L = 16
NUM_TILES = 16
NUM_WORKERS = 32
PASS_COLS = FEAT_DIM // NUM_WORKERS
CGROUPS = PASS_COLS // L
R = 128
RB = R // 8
NCHUNKS = N // R
LGROUPS = R // L
CBLOCKS = FEAT_DIM // 128
BLEND_CHUNK = 200
BLEND_RB = BLEND_CHUNK // 8
BLEND_CHUNKS = NUM_CLASSES // BLEND_CHUNK

_mesh = plsc.VectorSubcoreMesh(core_axis_name="c", subcore_axis_name="s")


@functools.partial(
    pl.kernel,
    out_type=(
        jax.ShapeDtypeStruct((NUM_CLASSES // 8, CBLOCKS, 8, 128), jnp.float32),
        jax.ShapeDtypeStruct((NUM_CLASSES // 8, CBLOCKS, 8, 128), jnp.float32),
    ),
    mesh=_mesh,
    compiler_params=pltpu.CompilerParams(
        use_tc_tiling_on_sc=False, needs_layout_passes=False
    ),
    scratch_types=[
        pltpu.VMEM((NUM_CLASSES, PASS_COLS), jnp.float32),
        pltpu.VMEM((NUM_CLASSES, L), jnp.float32),
        pltpu.VMEM((RB, 8, PASS_COLS), jnp.float32),
        pltpu.VMEM((RB, 8, PASS_COLS), jnp.float32),
        pltpu.VMEM((N,), jnp.int32),
        pltpu.VMEM((BLEND_RB, 8, PASS_COLS), jnp.float32),
        pltpu.SemaphoreType.DMA,
        pltpu.SemaphoreType.DMA,
    ],
)
def _cma_update(
    rgb_hbm, rgb_lab_hbm, vis_mem_hbm, ir_hbm, ir_lab_hbm, ir_mem_hbm,
    vis_out_hbm, ir_out_hbm,
    acc, cnt, feat_a, feat_b, labv, memv, sem_a, sem_b,
):
    c = lax.axis_index("c")
    s = lax.axis_index("s")
    w = c * NUM_TILES + s
    cb = w // 2
    hoff = (w % 2) * PASS_COLS
    iota = lax.iota(jnp.int32, L)
    ones = jnp.ones((L,), jnp.float32)

    def process_chunk(buf, i):
        @plsc.parallel_loop(0, LGROUPS)
        def _(g):
            lvec = labv[pl.ds((i * LGROUPS + g) * L, L)]

            def ld(j):
                lab_bc = jnp.take_along_axis(
                    lvec, jnp.full((L,), j, jnp.int32), axis=0
                )
                b = (g * L + j) // 8
                r = j % 8
                vals = [buf[b, r, pl.ds(cg * L, L)] for cg in range(CGROUPS)]
                return lab_bc, vals

            def st(lab_bc, vals):
                for cg in range(CGROUPS):
                    plsc.addupdate_scatter(
                        acc, [lab_bc, cg * L + iota], vals[cg]
                    )
                plsc.addupdate_scatter(cnt, [lab_bc, iota], ones)

            p0 = ld(0)
            p1 = ld(1)
            for j in range(2, L):
                st(*p0)
                p0 = p1
                p1 = ld(j)
            st(*p0)
            st(*p1)

    for feats_hbm, lab_hbm, mem_hbm, out_hbm in (
        (rgb_hbm, rgb_lab_hbm, vis_mem_hbm, vis_out_hbm),
        (ir_hbm, ir_lab_hbm, ir_mem_hbm, ir_out_hbm),
    ):
        @pl.loop(0, NUM_CLASSES)
        def _(r):
            for cg in range(CGROUPS):
                acc[r, pl.ds(cg * L, L)] = jnp.zeros((L,), jnp.float32)
            cnt[r, :] = jnp.zeros((L,), jnp.float32)

        pltpu.sync_copy(lab_hbm, labv)

        def rows(i):
            return feats_hbm.at[pl.ds(i * RB, RB), cb, :, pl.ds(hoff, PASS_COLS)]

        pltpu.async_copy(rows(0), feat_a, sem_a)

        @pl.loop(0, NCHUNKS // 2)
        def _(i):
            i0 = 2 * i
            i1 = i0 + 1
            pltpu.make_async_copy(rows(i0), feat_a, sem_a).wait()
            pltpu.async_copy(rows(i1), feat_b, sem_b)
            process_chunk(feat_a, i0)
            pltpu.make_async_copy(rows(i1), feat_b, sem_b).wait()

            @pl.when(i1 + 1 < NCHUNKS)
            def _():
                pltpu.async_copy(rows(i1 + 1), feat_a, sem_a)

            process_chunk(feat_b, i1)

        @pl.loop(0, BLEND_CHUNKS)
        def _(ch):
            pltpu.sync_copy(
                mem_hbm.at[
                    pl.ds(ch * BLEND_RB, BLEND_RB), cb, :, pl.ds(hoff, PASS_COLS)
                ],
                memv,
            )

            @pl.loop(0, BLEND_CHUNK)
            def _(jj):
                cls = ch * BLEND_CHUNK + jj
                b = jj // 8
                r = jj % 8
                cvec = cnt[cls, :]
                present = cvec > 0.0
                avec = jnp.where(present, SIGMA / jnp.maximum(cvec, 1.0), 0.0)
                bvec = jnp.where(present, 1.0 - SIGMA, 1.0)
                for cg in range(CGROUPS):
                    sl = pl.ds(cg * L, L)
                    memv[b, r, sl] = bvec * memv[b, r, sl] + avec * acc[cls, sl]

            pltpu.sync_copy(
                memv,
                out_hbm.at[
                    pl.ds(ch * BLEND_RB, BLEND_RB), cb, :, pl.ds(hoff, PASS_COLS)
                ],
            )


def kernel(rgb_feats, ir_feats, rgb_labels, ir_labels, vis_memory, ir_memory):
    def to4(x, nrows):
        return x.reshape(nrows // 8, 8, CBLOCKS, 128).transpose(0, 2, 1, 3)

    def from4(x, nrows):
        return x.transpose(0, 2, 1, 3).reshape(nrows, FEAT_DIM)

    vis_out, ir_out = _cma_update(
        to4(rgb_feats, N),
        rgb_labels.astype(jnp.int32),
        to4(vis_memory, NUM_CLASSES),
        to4(ir_feats, N),
        ir_labels.astype(jnp.int32),
        to4(ir_memory, NUM_CLASSES),
    )
    return (from4(vis_out, NUM_CLASSES), from4(ir_out, NUM_CLASSES))
